# Initial kernel scaffold; baseline (speedup 1.0000x reference)
import functools

import jax
import jax.numpy as jnp
from jax import lax
from jax.experimental import pallas as pl
from jax.experimental.pallas import tpu as pltpu
from jax.experimental.pallas import tpu_sc as plsc

N = 10000
E = 320000
D = 128
CD = 3
ED = 4
H = 128

TW = 144
NPAD = 10240
NSC = 2
NSUB = 16
NW = NSC * NSUB
EPW = E // NW
CHUNK = 80
NCH = EPW // CHUNK
ROWS_PER_SUB = NPAD // NSUB

_f32 = jnp.float32


def _prep_body(feats_ref, coordp_ref, we1i_ref, we1j_ref, tdst_ref, tsrc_ref):
    feats = feats_ref[...]
    coordp = coordp_ref[...]
    p = jnp.dot(feats, we1i_ref[...], preferred_element_type=_f32)
    q = jnp.dot(feats, we1j_ref[...], preferred_element_type=_f32)
    tdst_ref[...] = jnp.concatenate([p, coordp], axis=1)
    tsrc_ref[...] = jnp.concatenate([q, coordp], axis=1)


def _prep_call(feats, coordp, we1i, we1j):
    bn = 1000
    return pl.pallas_call(
        _prep_body,
        grid=(N // bn,),
        in_specs=[
            pl.BlockSpec((bn, D), lambda i: (i, 0)),
            pl.BlockSpec((bn, TW - D), lambda i: (i, 0)),
            pl.BlockSpec((D, H), lambda i: (0, 0)),
            pl.BlockSpec((D, H), lambda i: (0, 0)),
        ],
        out_specs=[
            pl.BlockSpec((bn, TW), lambda i: (i, 0)),
            pl.BlockSpec((bn, TW), lambda i: (i, 0)),
        ],
        out_shape=[
            jax.ShapeDtypeStruct((N, TW), _f32),
            jax.ShapeDtypeStruct((N, TW), _f32),
        ],
    )(feats, coordp, we1i, we1j)


def _gather_body(tsrc_hbm, tdst_hbm, src_hbm, dst_hbm, gsrc_hbm, gdst_hbm,
                 idxs_v, idxd_v, rows_v, sem):
    c = lax.axis_index("c")
    s = lax.axis_index("s")
    wid = s * NSC + c
    pltpu.sync_copy(src_hbm.at[wid], idxs_v)
    pltpu.sync_copy(dst_hbm.at[wid], idxd_v)
    base = wid * EPW

    @pl.loop(0, NCH)
    def _(j):
        off = base + j * CHUNK
        pltpu.async_copy(tsrc_hbm.at[idxs_v.at[j]], rows_v, sem).wait()
        pltpu.sync_copy(rows_v, gsrc_hbm.at[pl.ds(off, CHUNK)])
        pltpu.async_copy(tdst_hbm.at[idxd_v.at[j]], rows_v, sem).wait()
        pltpu.sync_copy(rows_v, gdst_hbm.at[pl.ds(off, CHUNK)])


def _gather_call(tsrc, tdst, src_r, dst_r):
    mesh = plsc.VectorSubcoreMesh(core_axis_name="c", subcore_axis_name="s")
    k = functools.partial(
        pl.kernel,
        out_type=(
            jax.ShapeDtypeStruct((E, TW), _f32),
            jax.ShapeDtypeStruct((E, TW), _f32),
        ),
        mesh=mesh,
        scratch_types=[
            pltpu.VMEM((NCH, CHUNK), jnp.int32),
            pltpu.VMEM((NCH, CHUNK), jnp.int32),
            pltpu.VMEM((CHUNK, TW), _f32),
            pltpu.SemaphoreType.DMA,
        ],
    )(_gather_body)
    return k(tsrc, tdst, src_r, dst_r)


def _edge_body(gsrc_ref, gdst_ref, ea_ref, wa_ref, wn_ref, be1_ref,
               we2_ref, be2_ref, wc1_ref, bc1_ref, wc2_ref, bc2_ref, v_ref):
    gsrc = gsrc_ref[...]
    gdst = gdst_ref[...]
    dc = gsrc[:, D:D + CD] - gdst[:, D:D + CD]
    dn2 = jnp.sum(dc * dc, axis=1, keepdims=True)
    pre1 = (gdst[:, :D] + gsrc[:, :D]
            + jnp.dot(ea_ref[...], wa_ref[...], preferred_element_type=_f32)
            + dn2 * wn_ref[...] + be1_ref[...])
    m = jax.nn.silu(pre1)
    m2 = jax.nn.silu(jnp.dot(m, we2_ref[...], preferred_element_type=_f32)
                     + be2_ref[...])
    t = jax.nn.silu(jnp.dot(m2, wc1_ref[...], preferred_element_type=_f32)
                    + bc1_ref[...])
    cw = jnp.sum(t * wc2_ref[...], axis=1, keepdims=True) + bc2_ref[...]
    be = gsrc.shape[0]
    v_ref[...] = jnp.concatenate(
        [m2, dc * cw, jnp.ones((be, 1), _f32),
         jnp.zeros((be, TW - D - CD - 1), _f32)],
        axis=1)


def _edge_call(gsrc, gdst, edge_attr, wa, wn, be1, We2, be2, Wc1, bc1, wc2, bc2):
    be = 2000
    full = lambda i: (0, 0)
    return pl.pallas_call(
        _edge_body,
        grid=(E // be,),
        in_specs=[
            pl.BlockSpec((be, TW), lambda i: (i, 0)),
            pl.BlockSpec((be, TW), lambda i: (i, 0)),
            pl.BlockSpec((be, ED), lambda i: (i, 0)),
            pl.BlockSpec((ED, H), full),
            pl.BlockSpec((1, H), full),
            pl.BlockSpec((1, H), full),
            pl.BlockSpec((H, H), full),
            pl.BlockSpec((1, H), full),
            pl.BlockSpec((H, H), full),
            pl.BlockSpec((1, H), full),
            pl.BlockSpec((1, H), full),
            pl.BlockSpec((1, 1), full),
        ],
        out_specs=pl.BlockSpec((be, TW), lambda i: (i, 0)),
        out_shape=jax.ShapeDtypeStruct((E, TW), _f32),
    )(gsrc, gdst, edge_attr, wa, wn, be1, We2, be2, Wc1, bc1, wc2, bc2)


def _scatter_body(v_hbm, dst_hbm, out_hbm, idx_v, val_v, acc_sh, sem):
    c = lax.axis_index("c")
    s = lax.axis_index("s")

    @pl.loop(0, CHUNK)
    def _(r):
        @pl.loop(0, TW, step=16)
        def _(l):
            val_v[r, pl.ds(l, 16)] = jnp.zeros((16,), _f32)

    @pl.loop(0, ROWS_PER_SUB, step=CHUNK)
    def _(z):
        pltpu.sync_copy(val_v, acc_sh.at[pl.ds(s * ROWS_PER_SUB + z, CHUNK)])

    plsc.subcore_barrier()

    wid = s * NSC + c
    base = (c * NSUB + s) * EPW
    pltpu.sync_copy(dst_hbm.at[wid], idx_v)

    @pl.loop(0, NCH)
    def _(j):
        pltpu.sync_copy(v_hbm.at[pl.ds(base + j * CHUNK, CHUNK)], val_v)
        pltpu.sync_copy(val_v, acc_sh.at[idx_v.at[j]], add=True)

    plsc.subcore_barrier()
    pltpu.sync_copy(acc_sh.at[pl.ds(s * ROWS_PER_SUB, ROWS_PER_SUB)],
                    out_hbm.at[c, pl.ds(s * ROWS_PER_SUB, ROWS_PER_SUB)])


def _scatter_call(vals, dst_sc):
    mesh = plsc.VectorSubcoreMesh(core_axis_name="c", subcore_axis_name="s")
    k = functools.partial(
        pl.kernel,
        out_type=jax.ShapeDtypeStruct((NSC, NPAD, TW), _f32),
        mesh=mesh,
        scratch_types=[
            pltpu.VMEM((NCH, CHUNK), jnp.int32),
            pltpu.VMEM((CHUNK, TW), _f32),
            pltpu.VMEM_SHARED((NPAD, TW), _f32),
            pltpu.SemaphoreType.DMA,
        ],
    )(_scatter_body)
    return k(vals, dst_sc)


def _node_body(a0_ref, a1_ref, feats_ref, coord_ref, wl1f_ref, wl1m_ref,
               bl1_ref, wl2_ref, bl2_ref, out_ref):
    s = a0_ref[...] + a1_ref[...]
    feats = feats_ref[...]
    inv = 1.0 / jnp.maximum(s[:, D + CD:D + CD + 1], 1.0)
    m_i = s[:, :D] * inv
    mhat = s[:, D:D + CD] * inv
    coord_out = coord_ref[...] + mhat
    h = jax.nn.silu(jnp.dot(feats, wl1f_ref[...], preferred_element_type=_f32)
                    + jnp.dot(m_i, wl1m_ref[...], preferred_element_type=_f32)
                    + bl1_ref[...])
    lat = feats + jnp.dot(h, wl2_ref[...], preferred_element_type=_f32) \
        + bl2_ref[...]
    out_ref[...] = jnp.concatenate([coord_out, lat], axis=1)


def _node_call(a0, a1, feats, coord, wl1f, wl1m, bl1, Wl2, bl2):
    bn = 1000
    full = lambda i: (0, 0)
    return pl.pallas_call(
        _node_body,
        grid=(N // bn,),
        in_specs=[
            pl.BlockSpec((bn, TW), lambda i: (i, 0)),
            pl.BlockSpec((bn, TW), lambda i: (i, 0)),
            pl.BlockSpec((bn, D), lambda i: (i, 0)),
            pl.BlockSpec((bn, CD), lambda i: (i, 0)),
            pl.BlockSpec((D, H), full),
            pl.BlockSpec((H, H), full),
            pl.BlockSpec((1, H), full),
            pl.BlockSpec((H, D), full),
            pl.BlockSpec((1, D), full),
        ],
        out_specs=pl.BlockSpec((bn, CD + D), lambda i: (i, 0)),
        out_shape=jax.ShapeDtypeStruct((N, CD + D), _f32),
    )(a0, a1, feats, coord, wl1f, wl1m, bl1, Wl2, bl2)


def kernel(x, edge_index, edge_attr,
           We1, be1, We2, be2,
           Wc1, bc1, Wc2, bc2,
           Wl1, bl1, Wl2, bl2):
    coord = x[:, :CD]
    feats = x[:, CD:]
    coordp = jnp.pad(coord, ((0, 0), (0, TW - D - CD)))
    src_r = edge_index[0].reshape(NW, NCH, CHUNK)
    dst_r = edge_index[1].reshape(NW, NCH, CHUNK)

    we1i = We1[:D]
    we1j = We1[D:2 * D]
    wa = We1[2 * D:2 * D + ED]
    wn = We1[2 * D + ED:].reshape(1, H)

    tdst, tsrc = _prep_call(feats, coordp, we1i, we1j)
    gsrc, gdst = _gather_call(tsrc, tdst, src_r, dst_r)
    vals = _edge_call(gsrc, gdst, edge_attr, wa, wn, be1.reshape(1, H),
                      We2, be2.reshape(1, H), Wc1, bc1.reshape(1, H),
                      Wc2.reshape(1, H), bc2.reshape(1, 1))
    partials = _scatter_call(vals, dst_r)
    out = _node_call(partials[0, :N], partials[1, :N], feats, coord,
                     Wl1[:D], Wl1[D:], bl1.reshape(1, H), Wl2, bl2.reshape(1, D))
    return out


# speedup vs baseline: 2.4379x; 2.4379x over previous
import functools

import jax
import jax.numpy as jnp
from jax import lax
from jax.experimental import pallas as pl
from jax.experimental.pallas import tpu as pltpu
from jax.experimental.pallas import tpu_sc as plsc

N = 10000
E = 320000
D = 128
CD = 3
ED = 4
H = 128

TW = 144
NPAD = 10240
NSC = 2
NSUB = 16
NW = NSC * NSUB
EPW = E // NW
CHUNK = 80
NCH = EPW // CHUNK
ROWS_PER_SUB = NPAD // NSUB

_f32 = jnp.float32


def _prep_body(feats_ref, coordp_ref, we1i_ref, we1j_ref, tdst_ref, tsrc_ref):
    feats = feats_ref[...]
    coordp = coordp_ref[...]
    p = jnp.dot(feats, we1i_ref[...], preferred_element_type=_f32)
    q = jnp.dot(feats, we1j_ref[...], preferred_element_type=_f32)
    tdst_ref[...] = jnp.concatenate([p, coordp], axis=1)
    tsrc_ref[...] = jnp.concatenate([q, coordp], axis=1)


def _prep_call(feats, coordp, we1i, we1j):
    bn = 1000
    return pl.pallas_call(
        _prep_body,
        grid=(N // bn,),
        in_specs=[
            pl.BlockSpec((bn, D), lambda i: (i, 0)),
            pl.BlockSpec((bn, TW - D), lambda i: (i, 0)),
            pl.BlockSpec((D, H), lambda i: (0, 0)),
            pl.BlockSpec((D, H), lambda i: (0, 0)),
        ],
        out_specs=[
            pl.BlockSpec((bn, TW), lambda i: (i, 0)),
            pl.BlockSpec((bn, TW), lambda i: (i, 0)),
        ],
        out_shape=[
            jax.ShapeDtypeStruct((N, TW), _f32),
            jax.ShapeDtypeStruct((N, TW), _f32),
        ],
    )(feats, coordp, we1i, we1j)


def _gather_body(tsrc_hbm, tdst_hbm, src_hbm, dst_hbm, gsrc_hbm, gdst_hbm,
                 idxs_v, idxd_v, rows_v, sem):
    c = lax.axis_index("c")
    s = lax.axis_index("s")
    wid = s * NSC + c
    pltpu.sync_copy(src_hbm.at[wid], idxs_v)
    pltpu.sync_copy(dst_hbm.at[wid], idxd_v)
    base = wid * EPW

    @pl.loop(0, NCH)
    def _(j):
        off = base + j * CHUNK
        pltpu.async_copy(tsrc_hbm.at[idxs_v.at[j]], rows_v, sem).wait()
        pltpu.sync_copy(rows_v, gsrc_hbm.at[pl.ds(off, CHUNK)])
        pltpu.async_copy(tdst_hbm.at[idxd_v.at[j]], rows_v, sem).wait()
        pltpu.sync_copy(rows_v, gdst_hbm.at[pl.ds(off, CHUNK)])


def _gather_call(tsrc, tdst, src_r, dst_r):
    mesh = plsc.VectorSubcoreMesh(core_axis_name="c", subcore_axis_name="s")
    k = functools.partial(
        pl.kernel,
        out_type=(
            jax.ShapeDtypeStruct((E, TW), _f32),
            jax.ShapeDtypeStruct((E, TW), _f32),
        ),
        mesh=mesh,
        scratch_types=[
            pltpu.VMEM((NCH, CHUNK), jnp.int32),
            pltpu.VMEM((NCH, CHUNK), jnp.int32),
            pltpu.VMEM((CHUNK, TW), _f32),
            pltpu.SemaphoreType.DMA,
        ],
        compiler_params=pltpu.CompilerParams(use_tc_tiling_on_sc=False),
    )(_gather_body)
    return k(tsrc, tdst, src_r, dst_r)


def _edge_body(gsrc_ref, gdst_ref, ea_ref, wa_ref, wn_ref, be1_ref,
               we2_ref, be2_ref, wc1_ref, bc1_ref, wc2_ref, bc2_ref, v_ref):
    gsrc = gsrc_ref[...]
    gdst = gdst_ref[...]
    dc = gsrc[:, D:D + CD] - gdst[:, D:D + CD]
    dn2 = jnp.sum(dc * dc, axis=1, keepdims=True)
    pre1 = (gdst[:, :D] + gsrc[:, :D]
            + jnp.dot(ea_ref[...], wa_ref[...], preferred_element_type=_f32)
            + dn2 * wn_ref[...] + be1_ref[...])
    m = jax.nn.silu(pre1)
    m2 = jax.nn.silu(jnp.dot(m, we2_ref[...], preferred_element_type=_f32)
                     + be2_ref[...])
    t = jax.nn.silu(jnp.dot(m2, wc1_ref[...], preferred_element_type=_f32)
                    + bc1_ref[...])
    cw = jnp.sum(t * wc2_ref[...], axis=1, keepdims=True) + bc2_ref[...]
    be = gsrc.shape[0]
    v_ref[...] = jnp.concatenate(
        [m2, dc * cw, jnp.ones((be, 1), _f32),
         jnp.zeros((be, TW - D - CD - 1), _f32)],
        axis=1)


def _edge_call(gsrc, gdst, edge_attr, wa, wn, be1, We2, be2, Wc1, bc1, wc2, bc2):
    be = 2000
    full = lambda i: (0, 0)
    return pl.pallas_call(
        _edge_body,
        grid=(E // be,),
        in_specs=[
            pl.BlockSpec((be, TW), lambda i: (i, 0)),
            pl.BlockSpec((be, TW), lambda i: (i, 0)),
            pl.BlockSpec((be, ED), lambda i: (i, 0)),
            pl.BlockSpec((ED, H), full),
            pl.BlockSpec((1, H), full),
            pl.BlockSpec((1, H), full),
            pl.BlockSpec((H, H), full),
            pl.BlockSpec((1, H), full),
            pl.BlockSpec((H, H), full),
            pl.BlockSpec((1, H), full),
            pl.BlockSpec((1, H), full),
            pl.BlockSpec((1, 1), full),
        ],
        out_specs=pl.BlockSpec((be, TW), lambda i: (i, 0)),
        out_shape=jax.ShapeDtypeStruct((E, TW), _f32),
    )(gsrc, gdst, edge_attr, wa, wn, be1, We2, be2, Wc1, bc1, wc2, bc2)


def _scatter_body(v_hbm, dst_hbm, out_hbm, idx_v, val_v, acc_sh, sem):
    c = lax.axis_index("c")
    s = lax.axis_index("s")

    @pl.loop(0, CHUNK)
    def _(r):
        @pl.loop(0, TW, step=16)
        def _(l):
            val_v[r, pl.ds(l, 16)] = jnp.zeros((16,), _f32)

    @pl.loop(0, ROWS_PER_SUB, step=CHUNK)
    def _(z):
        pltpu.sync_copy(val_v, acc_sh.at[pl.ds(s * ROWS_PER_SUB + z, CHUNK)])

    plsc.subcore_barrier()

    wid = s * NSC + c
    base = wid * EPW
    pltpu.sync_copy(dst_hbm.at[wid], idx_v)

    @pl.loop(0, NCH)
    def _(j):
        pltpu.sync_copy(v_hbm.at[pl.ds(base + j * CHUNK, CHUNK)], val_v)
        pltpu.sync_copy(val_v, acc_sh.at[idx_v.at[j]], add=True)

    plsc.subcore_barrier()
    pltpu.sync_copy(acc_sh.at[pl.ds(s * ROWS_PER_SUB, ROWS_PER_SUB)],
                    out_hbm.at[c, pl.ds(s * ROWS_PER_SUB, ROWS_PER_SUB)])


def _scatter_call(vals, dst_sc):
    mesh = plsc.VectorSubcoreMesh(core_axis_name="c", subcore_axis_name="s")
    k = functools.partial(
        pl.kernel,
        out_type=jax.ShapeDtypeStruct((NSC, NPAD, TW), _f32),
        mesh=mesh,
        scratch_types=[
            pltpu.VMEM((NCH, CHUNK), jnp.int32),
            pltpu.VMEM((CHUNK, TW), _f32),
            pltpu.VMEM_SHARED((NPAD, TW), _f32),
            pltpu.SemaphoreType.DMA,
        ],
        compiler_params=pltpu.CompilerParams(use_tc_tiling_on_sc=False),
    )(_scatter_body)
    return k(vals, dst_sc)


def _node_body(a0_ref, a1_ref, feats_ref, coord_ref, wl1f_ref, wl1m_ref,
               bl1_ref, wl2_ref, bl2_ref, out_ref):
    s = a0_ref[...] + a1_ref[...]
    feats = feats_ref[...]
    inv = 1.0 / jnp.maximum(s[:, D + CD:D + CD + 1], 1.0)
    m_i = s[:, :D] * inv
    mhat = s[:, D:D + CD] * inv
    coord_out = coord_ref[...] + mhat
    h = jax.nn.silu(jnp.dot(feats, wl1f_ref[...], preferred_element_type=_f32)
                    + jnp.dot(m_i, wl1m_ref[...], preferred_element_type=_f32)
                    + bl1_ref[...])
    lat = feats + jnp.dot(h, wl2_ref[...], preferred_element_type=_f32) \
        + bl2_ref[...]
    out_ref[...] = jnp.concatenate([coord_out, lat], axis=1)


def _node_call(a0, a1, feats, coord, wl1f, wl1m, bl1, Wl2, bl2):
    bn = 1000
    full = lambda i: (0, 0)
    return pl.pallas_call(
        _node_body,
        grid=(N // bn,),
        in_specs=[
            pl.BlockSpec((bn, TW), lambda i: (i, 0)),
            pl.BlockSpec((bn, TW), lambda i: (i, 0)),
            pl.BlockSpec((bn, D), lambda i: (i, 0)),
            pl.BlockSpec((bn, CD), lambda i: (i, 0)),
            pl.BlockSpec((D, H), full),
            pl.BlockSpec((H, H), full),
            pl.BlockSpec((1, H), full),
            pl.BlockSpec((H, D), full),
            pl.BlockSpec((1, D), full),
        ],
        out_specs=pl.BlockSpec((bn, CD + D), lambda i: (i, 0)),
        out_shape=jax.ShapeDtypeStruct((N, CD + D), _f32),
    )(a0, a1, feats, coord, wl1f, wl1m, bl1, Wl2, bl2)


def kernel(x, edge_index, edge_attr,
           We1, be1, We2, be2,
           Wc1, bc1, Wc2, bc2,
           Wl1, bl1, Wl2, bl2):
    coord = x[:, :CD]
    feats = x[:, CD:]
    coordp = jnp.pad(coord, ((0, 0), (0, TW - D - CD)))
    src_r = edge_index[0].reshape(NW, NCH, CHUNK)
    dst_r = edge_index[1].reshape(NW, NCH, CHUNK)

    we1i = We1[:D]
    we1j = We1[D:2 * D]
    wa = We1[2 * D:2 * D + ED]
    wn = We1[2 * D + ED:].reshape(1, H)

    tdst, tsrc = _prep_call(feats, coordp, we1i, we1j)
    gsrc, gdst = _gather_call(tsrc, tdst, src_r, dst_r)
    vals = _edge_call(gsrc, gdst, edge_attr, wa, wn, be1.reshape(1, H),
                      We2, be2.reshape(1, H), Wc1, bc1.reshape(1, H),
                      Wc2.reshape(1, H), bc2.reshape(1, 1))
    partials = _scatter_call(vals, dst_r)
    out = _node_call(partials[0, :N], partials[1, :N], feats, coord,
                     Wl1[:D], Wl1[D:], bl1.reshape(1, H), Wl2, bl2.reshape(1, D))
    return out
